# baseline (speedup 1.0000x reference)
import functools

import jax
import jax.numpy as jnp
from jax import lax
from jax.experimental import pallas as pl
from jax.experimental.pallas import tpu as pltpu
from jax.experimental.pallas import tpu_sc as plsc

D = 128
C = 128
NC = 2
NS = 16
NW = NC * NS
L = 16
NBUF = 2


def _decode(z, src, dst, cpw):
    e_pad = src.shape[0]
    mesh = plsc.VectorSubcoreMesh(core_axis_name="c", subcore_axis_name="s")

    @functools.partial(
        pl.kernel,
        mesh=mesh,
        compiler_params=pltpu.CompilerParams(needs_layout_passes=False),
        out_type=jax.ShapeDtypeStruct((e_pad,), jnp.float32),
        scratch_types=(
            [pltpu.VMEM((C,), jnp.int32) for _ in range(2 * NBUF)]
            + [pltpu.VMEM((C, D), jnp.float32) for _ in range(2 * NBUF)]
            + [pltpu.VMEM((C,), jnp.float32) for _ in range(NBUF)]
            + [pltpu.SemaphoreType.DMA for _ in range(3 * NBUF)]
        ),
    )
    def kern(z_hbm, src_hbm, dst_hbm, out_hbm,
             sidx0, sidx1, didx0, didx1, sr0, sr1, dr0, dr1, ov0, ov1,
             gs0, gs1, gd0, gd1, os0, os1):
        sidx = (sidx0, sidx1)
        didx = (didx0, didx1)
        srows = (sr0, sr1)
        drows = (dr0, dr1)
        outv = (ov0, ov1)
        gsem = (gs0, gs1)
        dsem = (gd0, gd1)
        osem = (os0, os1)

        wid = lax.axis_index("s") * NC + lax.axis_index("c")
        base0 = wid * cpw * C
        lane = lax.broadcasted_iota(jnp.int32, (L,), 0)

        def stage(j, b):
            off = base0 + j * C
            pltpu.sync_copy(src_hbm.at[pl.ds(off, C)], sidx[b])
            pltpu.sync_copy(dst_hbm.at[pl.ds(off, C)], didx[b])
            pltpu.async_copy(z_hbm.at[sidx[b]], srows[b], gsem[b])
            pltpu.async_copy(z_hbm.at[didx[b]], drows[b], dsem[b])

        def compute(b):
            def group_body(g, carry2):
                rows = lane + g * L

                def kbody(kk, acc):
                    cols = jnp.full((L,), kk, jnp.int32)
                    a = plsc.load_gather(srows[b], [rows, cols])
                    bb = plsc.load_gather(drows[b], [rows, cols])
                    return acc + a * bb

                acc = lax.fori_loop(0, D, kbody,
                                    jnp.zeros((L,), jnp.float32), unroll=32)
                outv[b][pl.ds(g * L, L)] = acc
                return carry2

            lax.fori_loop(0, C // L, group_body, 0)

        for b in range(NBUF):
            stage(b, b)

        def loop_body(i, carry):
            for b in range(NBUF):
                j = i * NBUF + b
                pltpu.make_async_copy(z_hbm.at[sidx[b]], srows[b],
                                      gsem[b]).wait()
                pltpu.make_async_copy(z_hbm.at[didx[b]], drows[b],
                                      dsem[b]).wait()

                @pl.when(j >= NBUF)
                def _():
                    pltpu.make_async_copy(outv[b],
                                          out_hbm.at[pl.ds(base0, C)],
                                          osem[b]).wait()

                compute(b)
                off = base0 + j * C
                pltpu.async_copy(outv[b], out_hbm.at[pl.ds(off, C)], osem[b])

                nj = j + NBUF

                @pl.when(nj < cpw)
                def _():
                    stage(nj, b)
            return carry

        lax.fori_loop(0, cpw // NBUF, loop_body, 0)

        for b in range(NBUF):
            pltpu.make_async_copy(outv[b], out_hbm.at[pl.ds(base0, C)],
                                  osem[b]).wait()

    return kern(z, src, dst)


def kernel(features, graph, pos_edge, neg_edge):
    z = features[-1]
    edge = jnp.concatenate([pos_edge, neg_edge], axis=-1)
    e = edge.shape[1]
    unit = NW * C * NBUF
    cpw = (-(-e // unit)) * NBUF
    e_pad = cpw * NW * C
    src = jnp.pad(edge[0], (0, e_pad - e))
    dst = jnp.pad(edge[1], (0, e_pad - e))
    out = _decode(z, src, dst, cpw)
    return out[:e]


# speedup vs baseline: 2.9082x; 2.9082x over previous
import functools

import jax
import jax.numpy as jnp
from jax import lax
from jax.experimental import pallas as pl
from jax.experimental.pallas import tpu as pltpu
from jax.experimental.pallas import tpu_sc as plsc

D = 128
C = 128
NC = 2
NS = 16
NW = NC * NS
L = 16
NBUF = 2


def _decode(z, src, dst, cpw):
    e_pad = src.shape[0]
    mesh = plsc.VectorSubcoreMesh(core_axis_name="c", subcore_axis_name="s")

    @functools.partial(
        pl.kernel,
        mesh=mesh,
        compiler_params=pltpu.CompilerParams(needs_layout_passes=False),
        out_type=jax.ShapeDtypeStruct((e_pad,), jnp.float32),
        scratch_types=(
            [pltpu.VMEM((C,), jnp.int32) for _ in range(2 * NBUF)]
            + [pltpu.VMEM((C, D), jnp.float32) for _ in range(2 * NBUF)]
            + [pltpu.VMEM((C,), jnp.float32) for _ in range(NBUF)]
            + [pltpu.SemaphoreType.DMA for _ in range(3 * NBUF)]
        ),
    )
    def kern(z_hbm, src_hbm, dst_hbm, out_hbm,
             sidx0, sidx1, didx0, didx1, sr0, sr1, dr0, dr1, ov0, ov1,
             gs0, gs1, gd0, gd1, os0, os1):
        sidx = (sidx0, sidx1)
        didx = (didx0, didx1)
        srows = (sr0, sr1)
        drows = (dr0, dr1)
        outv = (ov0, ov1)
        gsem = (gs0, gs1)
        dsem = (gd0, gd1)
        osem = (os0, os1)

        wid = lax.axis_index("s") * NC + lax.axis_index("c")
        base0 = wid * cpw * C
        lane = lax.broadcasted_iota(jnp.int32, (L,), 0)

        def stage(j, b):
            off = base0 + j * C
            pltpu.sync_copy(src_hbm.at[pl.ds(off, C)], sidx[b])
            pltpu.sync_copy(dst_hbm.at[pl.ds(off, C)], didx[b])
            pltpu.async_copy(z_hbm.at[sidx[b]], srows[b], gsem[b])
            pltpu.async_copy(z_hbm.at[didx[b]], drows[b], dsem[b])

        def compute(b):
            def group_body(g, carry2):
                def quad(q, res):
                    for i in range(4):
                        e = g * L + q * 4 + i
                        acc = (srows[b][e, pl.ds(0, L)]
                               * drows[b][e, pl.ds(0, L)])
                        for k8 in range(1, D // L):
                            a = srows[b][e, pl.ds(k8 * L, L)]
                            bb = drows[b][e, pl.ds(k8 * L, L)]
                            acc = acc + a * bb
                        res = jnp.where(lane == q * 4 + i, jnp.sum(acc), res)
                    return res

                res = lax.fori_loop(0, 4, quad, jnp.zeros((L,), jnp.float32))
                outv[b][pl.ds(g * L, L)] = res
                return carry2

            lax.fori_loop(0, C // L, group_body, 0)

        for b in range(NBUF):
            stage(b, b)

        def loop_body(i, carry):
            for b in range(NBUF):
                j = i * NBUF + b
                pltpu.make_async_copy(z_hbm.at[sidx[b]], srows[b],
                                      gsem[b]).wait()
                pltpu.make_async_copy(z_hbm.at[didx[b]], drows[b],
                                      dsem[b]).wait()

                @pl.when(j >= NBUF)
                def _():
                    pltpu.make_async_copy(outv[b],
                                          out_hbm.at[pl.ds(base0, C)],
                                          osem[b]).wait()

                compute(b)
                off = base0 + j * C
                pltpu.async_copy(outv[b], out_hbm.at[pl.ds(off, C)], osem[b])

                nj = j + NBUF

                @pl.when(nj < cpw)
                def _():
                    stage(nj, b)
            return carry

        lax.fori_loop(0, cpw // NBUF, loop_body, 0)

        for b in range(NBUF):
            pltpu.make_async_copy(outv[b], out_hbm.at[pl.ds(base0, C)],
                                  osem[b]).wait()

    return kern(z, src, dst)


def kernel(features, graph, pos_edge, neg_edge):
    z = features[-1]
    edge = jnp.concatenate([pos_edge, neg_edge], axis=-1)
    e = edge.shape[1]
    unit = NW * C * NBUF
    cpw = (-(-e // unit)) * NBUF
    e_pad = cpw * NW * C
    src = jnp.pad(edge[0], (0, e_pad - e))
    dst = jnp.pad(edge[1], (0, e_pad - e))
    out = _decode(z, src, dst, cpw)
    return out[:e]


# speedup vs baseline: 4.7830x; 1.6447x over previous
import functools

import jax
import jax.numpy as jnp
from jax import lax
from jax.experimental import pallas as pl
from jax.experimental.pallas import tpu as pltpu
from jax.experimental.pallas import tpu_sc as plsc

D = 128
C = 128
NC = 2
NS = 16
L = 16
NBUF = 2
FRAC1 = 0.30


def _decode(z, src, dst, cpw0, cpw1):
    e_pad = src.shape[0]
    mesh = plsc.VectorSubcoreMesh(core_axis_name="c", subcore_axis_name="s")

    @functools.partial(
        pl.kernel,
        mesh=mesh,
        compiler_params=pltpu.CompilerParams(needs_layout_passes=False),
        out_type=jax.ShapeDtypeStruct((e_pad,), jnp.float32),
        scratch_types=(
            [pltpu.VMEM((C,), jnp.int32) for _ in range(2 * NBUF)]
            + [pltpu.VMEM((C, D), jnp.float32) for _ in range(2 * NBUF)]
            + [pltpu.VMEM((C,), jnp.float32) for _ in range(NBUF)]
            + [pltpu.SemaphoreType.DMA for _ in range(3 * NBUF)]
        ),
    )
    def kern(z_hbm, src_hbm, dst_hbm, out_hbm,
             sidx0, sidx1, didx0, didx1, sr0, sr1, dr0, dr1, ov0, ov1,
             gs0, gs1, gd0, gd1, os0, os1):
        sidx = (sidx0, sidx1)
        didx = (didx0, didx1)
        srows = (sr0, sr1)
        drows = (dr0, dr1)
        outv = (ov0, ov1)
        gsem = (gs0, gs1)
        dsem = (gd0, gd1)
        osem = (os0, os1)

        c = lax.axis_index("c")
        s = lax.axis_index("s")
        cpw = jnp.where(c == 0, cpw0, cpw1)
        base_chunk = jnp.where(c == 0, s * cpw0, NS * cpw0 + s * cpw1)
        base0 = base_chunk * C
        lane = lax.broadcasted_iota(jnp.int32, (L,), 0)

        def stage(j, b):
            off = base0 + j * C
            pltpu.sync_copy(src_hbm.at[pl.ds(off, C)], sidx[b])
            pltpu.sync_copy(dst_hbm.at[pl.ds(off, C)], didx[b])
            pltpu.async_copy(z_hbm.at[sidx[b]], srows[b], gsem[b])
            pltpu.async_copy(z_hbm.at[didx[b]], drows[b], dsem[b])

        def compute(b):
            def group_body(g, carry2):
                def quad(q, res):
                    for i in range(4):
                        e = g * L + q * 4 + i
                        acc = (srows[b][e, pl.ds(0, L)]
                               * drows[b][e, pl.ds(0, L)])
                        for k8 in range(1, D // L):
                            a = srows[b][e, pl.ds(k8 * L, L)]
                            bb = drows[b][e, pl.ds(k8 * L, L)]
                            acc = acc + a * bb
                        res = jnp.where(lane == q * 4 + i, jnp.sum(acc), res)
                    return res

                res = lax.fori_loop(0, 4, quad, jnp.zeros((L,), jnp.float32))
                outv[b][pl.ds(g * L, L)] = res
                return carry2

            lax.fori_loop(0, C // L, group_body, 0)

        for b in range(NBUF):
            stage(b, b)

        def loop_body(i, carry):
            for b in range(NBUF):
                j = i * NBUF + b
                pltpu.make_async_copy(z_hbm.at[sidx[b]], srows[b],
                                      gsem[b]).wait()
                pltpu.make_async_copy(z_hbm.at[didx[b]], drows[b],
                                      dsem[b]).wait()

                @pl.when(j >= NBUF)
                def _():
                    pltpu.make_async_copy(outv[b],
                                          out_hbm.at[pl.ds(base0, C)],
                                          osem[b]).wait()

                compute(b)
                off = base0 + j * C
                pltpu.async_copy(outv[b], out_hbm.at[pl.ds(off, C)], osem[b])

                nj = j + NBUF

                @pl.when(nj < cpw)
                def _():
                    stage(nj, b)
            return carry

        lax.fori_loop(0, cpw // NBUF, loop_body, 0)

        for b in range(NBUF):
            pltpu.make_async_copy(outv[b], out_hbm.at[pl.ds(base0, C)],
                                  osem[b]).wait()

    return kern(z, src, dst)


def kernel(features, graph, pos_edge, neg_edge):
    z = features[-1]
    edge = jnp.concatenate([pos_edge, neg_edge], axis=-1)
    e = edge.shape[1]
    unit = NS * C * NBUF
    t = -(-e // unit) * NBUF
    cpw1 = max(NBUF, int(t * FRAC1 / NBUF) * NBUF)
    cpw0 = t - cpw1
    e_pad = NS * t * C
    src = jnp.pad(edge[0], (0, e_pad - e))
    dst = jnp.pad(edge[1], (0, e_pad - e))
    out = _decode(z, src, dst, cpw0, cpw1)
    return out[:e]
